# Initial kernel scaffold; baseline (speedup 1.0000x reference)
import jax
import jax.numpy as jnp
from jax.experimental import pallas as pl


def kernel(x, edge_index, batch, pre_lin_W, pre_lin_b, pre_W, pre_b, post_W, post_b, lin_W, lin_b, bn_g, bn_b, mlp_W1, mlp_b1, mlp_W2, mlp_b2, mlp_W3, mlp_b3):
    raise NotImplementedError("write your pallas kernel here")


# speedup vs baseline: 2.5416x; 2.5416x over previous
import functools

import jax
import jax.numpy as jnp
from jax.experimental import pallas as pl

_N = 50000
_E = 800000
_G = 512
_T = 5
_F = 75
_FO = 15
_L = 4

_NB = 512
_EB = 2048


def _mm_kernel(x_ref, w_ref, b_ref, o_ref):
    o_ref[...] = (
        jnp.dot(x_ref[...], w_ref[...], preferred_element_type=jnp.float32)
        + b_ref[...]
    )


def _rows_mm(x, w, b, block):
    n, k = x.shape
    m = w.shape[1]
    b2 = b.reshape(1, m)
    return pl.pallas_call(
        _mm_kernel,
        grid=(pl.cdiv(n, block),),
        in_specs=[
            pl.BlockSpec((block, k), lambda i: (i, 0)),
            pl.BlockSpec((k, m), lambda i: (0, 0)),
            pl.BlockSpec((1, m), lambda i: (0, 0)),
        ],
        out_specs=pl.BlockSpec((block, m), lambda i: (i, 0)),
        out_shape=jax.ShapeDtypeStruct((n, m), jnp.float32),
    )(x, w, b2)


def _start_kernel(prev_ref, sc_ref, sh_ref, w_ref, b_ref, h_ref, a_ref, *, relu):
    hh = prev_ref[...] * sc_ref[...] + sh_ref[...]
    if relu:
        hh = jnp.maximum(hh, 0.0)
    h_ref[...] = hh
    a_ref[...] = (
        jnp.dot(hh, w_ref[...], preferred_element_type=jnp.float32) + b_ref[...]
    )


def _layer_start(prev, scale, shift, wi, bi, relu):
    return pl.pallas_call(
        functools.partial(_start_kernel, relu=relu),
        grid=(pl.cdiv(_N, _NB),),
        in_specs=[
            pl.BlockSpec((_NB, _F), lambda i: (i, 0)),
            pl.BlockSpec((1, _F), lambda i: (0, 0)),
            pl.BlockSpec((1, _F), lambda i: (0, 0)),
            pl.BlockSpec((_F, _T * _F), lambda i: (0, 0)),
            pl.BlockSpec((1, _T * _F), lambda i: (0, 0)),
        ],
        out_specs=[
            pl.BlockSpec((_NB, _F), lambda i: (i, 0)),
            pl.BlockSpec((_NB, _T * _F), lambda i: (i, 0)),
        ],
        out_shape=[
            jax.ShapeDtypeStruct((_N, _F), jnp.float32),
            jax.ShapeDtypeStruct((_N, _T * _F), jnp.float32),
        ],
    )(prev, scale, shift, wi, bi)


def _agg_kernel(h_ref, a_ref, s_ref, q_ref, mn_ref, mx_ref, deg_ref, adl_ref,
                wj_ref, wh_ref, wa_ref, wb_ref, wc_ref, pb_ref, lw_ref, lb_ref,
                o_ref):
    deg = deg_ref[...]
    adl = adl_ref[...]
    cnt = jnp.maximum(deg, 1.0)
    has = deg > 0.0
    ldeg = jnp.log(cnt + 1.0)
    amp = ldeg / adl
    att = adl / ldeg

    h = h_ref[...]
    a = a_ref[...]
    mean_c = jnp.dot(s_ref[...] / cnt, wj_ref[...],
                     preferred_element_type=jnp.float32)
    mean = jnp.where(has, a + mean_c, 0.0)
    mean2_c = q_ref[...] / cnt
    std = jnp.sqrt(jax.nn.relu(mean2_c - mean_c * mean_c) + 1e-5)
    mn = jnp.where(has, a + mn_ref[...], 0.0)
    mx = jnp.where(has, a + mx_ref[...], 0.0)

    parts = []
    for t in range(_T):
        sl = slice(_F * t, _F * (t + 1))
        agg = jnp.concatenate(
            [mean[:, sl], mn[:, sl], mx[:, sl], std[:, sl]], axis=1)
        y = (
            jnp.dot(agg, wa_ref[t], preferred_element_type=jnp.float32)
            + amp * jnp.dot(agg, wb_ref[t], preferred_element_type=jnp.float32)
            + att * jnp.dot(agg, wc_ref[t], preferred_element_type=jnp.float32)
        )
        parts.append(y)
    out = (
        jnp.dot(h, wh_ref[...], preferred_element_type=jnp.float32)
        + pb_ref[...]
        + jnp.concatenate(parts, axis=1)
    )
    o_ref[...] = (
        jnp.dot(out, lw_ref[...], preferred_element_type=jnp.float32)
        + lb_ref[...]
    )


def _layer_aggregate(h, a, s, q, mn, mx, deg, adl, wj, wh, wa, wb, wc, pb,
                     lw, lb):
    tf = _T * _F
    return pl.pallas_call(
        _agg_kernel,
        grid=(pl.cdiv(_N, _NB),),
        in_specs=[
            pl.BlockSpec((_NB, _F), lambda i: (i, 0)),
            pl.BlockSpec((_NB, tf), lambda i: (i, 0)),
            pl.BlockSpec((_NB, _F), lambda i: (i, 0)),
            pl.BlockSpec((_NB, tf), lambda i: (i, 0)),
            pl.BlockSpec((_NB, tf), lambda i: (i, 0)),
            pl.BlockSpec((_NB, tf), lambda i: (i, 0)),
            pl.BlockSpec((_NB, 1), lambda i: (i, 0)),
            pl.BlockSpec((1, 1), lambda i: (0, 0)),
            pl.BlockSpec((_F, tf), lambda i: (0, 0)),
            pl.BlockSpec((_F, _T * _FO), lambda i: (0, 0)),
            pl.BlockSpec((_T, 4 * _F, _FO), lambda i: (0, 0, 0)),
            pl.BlockSpec((_T, 4 * _F, _FO), lambda i: (0, 0, 0)),
            pl.BlockSpec((_T, 4 * _F, _FO), lambda i: (0, 0, 0)),
            pl.BlockSpec((1, _T * _FO), lambda i: (0, 0)),
            pl.BlockSpec((_F, _F), lambda i: (0, 0)),
            pl.BlockSpec((1, _F), lambda i: (0, 0)),
        ],
        out_specs=pl.BlockSpec((_NB, _F), lambda i: (i, 0)),
        out_shape=jax.ShapeDtypeStruct((_N, _F), jnp.float32),
    )(h, a, s, q, mn, mx, deg, adl, wj, wh, wa, wb, wc, pb, lw, lb)


def _act_kernel(prev_ref, sc_ref, sh_ref, o_ref):
    o_ref[...] = jnp.maximum(prev_ref[...] * sc_ref[...] + sh_ref[...], 0.0)


def _bn_relu(prev, scale, shift):
    return pl.pallas_call(
        _act_kernel,
        grid=(pl.cdiv(_N, _NB),),
        in_specs=[
            pl.BlockSpec((_NB, _F), lambda i: (i, 0)),
            pl.BlockSpec((1, _F), lambda i: (0, 0)),
            pl.BlockSpec((1, _F), lambda i: (0, 0)),
        ],
        out_specs=pl.BlockSpec((_NB, _F), lambda i: (i, 0)),
        out_shape=jax.ShapeDtypeStruct((_N, _F), jnp.float32),
    )(prev, scale, shift)


def _mlp_kernel(g_ref, w1_ref, b1_ref, w2_ref, b2_ref, w3_ref, b3_ref, o_ref):
    g = jnp.maximum(
        jnp.dot(g_ref[...], w1_ref[...], preferred_element_type=jnp.float32)
        + b1_ref[...], 0.0)
    g = jnp.maximum(
        jnp.dot(g, w2_ref[...], preferred_element_type=jnp.float32)
        + b2_ref[...], 0.0)
    o_ref[...] = (
        jnp.dot(g, w3_ref[...], preferred_element_type=jnp.float32)
        + b3_ref[...]
    )


def _final_mlp(g, w1, b1, w2, b2, w3, b3):
    return pl.pallas_call(
        _mlp_kernel,
        grid=(1,),
        in_specs=[
            pl.BlockSpec((_G, _F), lambda i: (0, 0)),
            pl.BlockSpec((_F, 50), lambda i: (0, 0)),
            pl.BlockSpec((1, 50), lambda i: (0, 0)),
            pl.BlockSpec((50, 25), lambda i: (0, 0)),
            pl.BlockSpec((1, 25), lambda i: (0, 0)),
            pl.BlockSpec((25, 1), lambda i: (0, 0)),
            pl.BlockSpec((1, 1), lambda i: (0, 0)),
        ],
        out_specs=pl.BlockSpec((_G, 1), lambda i: (0, 0)),
        out_shape=jax.ShapeDtypeStruct((_G, 1), jnp.float32),
    )(g, w1, b1.reshape(1, 50), w2, b2.reshape(1, 25), w3, b3.reshape(1, 1))


def kernel(x, edge_index, batch, pre_lin_W, pre_lin_b, pre_W, pre_b, post_W,
           post_b, lin_W, lin_b, bn_g, bn_b, mlp_W1, mlp_b1, mlp_W2, mlp_b2,
           mlp_W3, mlp_b3):
    src = edge_index[0]
    dst = edge_index[1]

    deg = jnp.zeros((_N,), jnp.float32).at[dst].add(1.0)
    adl = jnp.mean(jnp.log(deg + 1.0)).reshape(1, 1)
    deg2 = deg.reshape(_N, 1)

    ones_f = jnp.ones((1, _F), jnp.float32)
    zeros_f = jnp.zeros((1, _F), jnp.float32)

    h = _rows_mm(x, pre_lin_W, pre_lin_b, _NB)

    prev = h
    scale, shift = ones_f, zeros_f
    for l in range(_L):
        wi = pre_W[l, :, :_F, :].transpose(1, 0, 2).reshape(_F, _T * _F)
        wj = pre_W[l, :, _F:, :].transpose(1, 0, 2).reshape(_F, _T * _F)
        bi = pre_b[l].reshape(1, _T * _F)
        wh = post_W[l, :, :_F, :].transpose(1, 0, 2).reshape(_F, _T * _FO)
        wa = post_W[l, :, _F:_F + 4 * _F, :]
        wb = post_W[l, :, _F + 4 * _F:_F + 8 * _F, :]
        wc = post_W[l, :, _F + 8 * _F:, :]
        pb = post_b[l].reshape(1, _T * _FO)

        h, a = _layer_start(prev, scale, shift, wi, bi, relu=(l > 0))

        hs = h[src]
        c = _rows_mm(hs, wj, jnp.zeros((_T * _F,), jnp.float32), _EB)

        s = jax.ops.segment_sum(hs, dst, num_segments=_N)
        q = jax.ops.segment_sum(c * c, dst, num_segments=_N)
        mn = jax.ops.segment_min(c, dst, num_segments=_N)
        mx = jax.ops.segment_max(c, dst, num_segments=_N)

        out = _layer_aggregate(h, a, s, q, mn, mx, deg2, adl, wj, wh, wa, wb,
                               wc, pb, lin_W[l], lin_b[l].reshape(1, _F))

        mu = jnp.mean(out, axis=0)
        var = jnp.var(out, axis=0)
        sc = bn_g[l] / jnp.sqrt(var + 1e-5)
        scale = sc.reshape(1, _F)
        shift = (bn_b[l] - mu * sc).reshape(1, _F)
        prev = out

    h_final = _bn_relu(prev, scale, shift)
    g = jax.ops.segment_sum(h_final, batch, num_segments=_G)
    return _final_mlp(g, mlp_W1, mlp_b1, mlp_W2, mlp_b2, mlp_W3, mlp_b3)
